# baseline (speedup 1.0000x reference)
import functools

import jax
import jax.numpy as jnp
from jax import lax
from jax.experimental import pallas as pl
from jax.experimental.pallas import tpu as pltpu
from jax.experimental.pallas import tpu_sc as plsc

_INFO = plsc.get_sparse_core_info()
_NC = _INFO.num_cores
_NS = _INFO.num_subcores
_NW = _NC * _NS
_CHUNK = 128


@functools.lru_cache(maxsize=None)
def _make_embed(B, V, D, dtype_name):
    dtype = jnp.dtype(dtype_name)
    K = B // (_CHUNK * _NW)
    mesh = plsc.VectorSubcoreMesh(core_axis_name="c", subcore_axis_name="s")

    @functools.partial(
        pl.kernel,
        mesh=mesh,
        out_type=jax.ShapeDtypeStruct((_NW * K, _CHUNK, D), dtype),
        scratch_types=[
            pltpu.VMEM((K, _CHUNK), jnp.int32),
            pltpu.VMEM((K, _CHUNK, D), dtype),
            pltpu.SemaphoreType.DMA,
        ],
        compiler_params=pltpu.CompilerParams(use_tc_tiling_on_sc=False),
    )
    def embed(idx_hbm, table_hbm, out_hbm, idx_v, rows_v, sem):
        wid = lax.axis_index("s") * _NC + lax.axis_index("c")
        base = wid * K
        pltpu.sync_copy(idx_hbm.at[pl.ds(base, K)], idx_v)
        copies = [
            pltpu.async_copy(table_hbm.at[idx_v.at[j]], rows_v.at[j], sem)
            for j in range(K)
        ]
        for c in copies:
            c.wait()
        pltpu.sync_copy(rows_v, out_hbm.at[pl.ds(base, K)])

    return embed


def kernel(x, weight):
    (B,) = x.shape
    V, D = weight.shape
    assert B % (_CHUNK * _NW) == 0
    idx = x.astype(jnp.int32).reshape(_NW * (B // (_CHUNK * _NW)), _CHUNK)
    out = _make_embed(B, V, D, weight.dtype.name)(idx, weight)
    return out.reshape(B, D)


# speedup vs baseline: 1.7123x; 1.7123x over previous
import functools

import jax
import jax.numpy as jnp
from jax import lax
from jax.experimental import pallas as pl
from jax.experimental.pallas import tpu as pltpu
from jax.experimental.pallas import tpu_sc as plsc

_INFO = plsc.get_sparse_core_info()
_NC = _INFO.num_cores
_NS = _INFO.num_subcores
_NW = _NC * _NS


@functools.lru_cache(maxsize=None)
def _make_embed(B, V, D):
    KC = B // _NW
    mesh = plsc.VectorSubcoreMesh(core_axis_name="c", subcore_axis_name="s")

    @functools.partial(
        pl.kernel,
        mesh=mesh,
        out_type=jax.ShapeDtypeStruct((B, D), jnp.float32),
        scratch_types=[
            pltpu.VMEM((KC,), jnp.int32),
            pltpu.VMEM((KC, D), jnp.float32),
            pltpu.SemaphoreType.DMA,
        ],
        compiler_params=pltpu.CompilerParams(
            use_tc_tiling_on_sc=True, needs_layout_passes=False
        ),
    )
    def embed(idx_hbm, table_hbm, out_hbm, idx_v, rows_v, sem):
        wid = lax.axis_index("s") * _NC + lax.axis_index("c")
        base = wid * KC
        pltpu.sync_copy(idx_hbm.at[pl.ds(base, KC)], idx_v)

        def group(g, carry):
            v = idx_v[pl.ds(g * 16, 16)]
            for lane in range(16):
                pltpu.async_copy(
                    table_hbm.at[pl.ds(v[lane], 1)],
                    rows_v.at[pl.ds(g * 16 + lane, 1)],
                    sem,
                )
            return carry

        lax.fori_loop(0, KC // 16, group, 0)
        pltpu.make_async_copy(table_hbm.at[pl.ds(0, KC)], rows_v, sem).wait()
        pltpu.sync_copy(rows_v, out_hbm.at[pl.ds(base, KC)])

    return embed


def kernel(x, weight):
    (B,) = x.shape
    V, D = weight.shape
    assert B % _NW == 0
    idx = x.astype(jnp.int32)
    out = _make_embed(B, V, D)(idx, weight)
    return out


# speedup vs baseline: 2.5253x; 1.4748x over previous
import functools

import jax
import jax.numpy as jnp
from jax import lax
from jax.experimental import pallas as pl
from jax.experimental.pallas import tpu as pltpu
from jax.experimental.pallas import tpu_sc as plsc

_INFO = plsc.get_sparse_core_info()
_NC = _INFO.num_cores
_NS = _INFO.num_subcores
_NW = _NC * _NS
_NBUF = 4
_LANES = 128


@functools.lru_cache(maxsize=None)
def _make_embed(B, V, D):
    KC = B // _NW
    mesh = plsc.VectorSubcoreMesh(core_axis_name="c", subcore_axis_name="s")

    @functools.partial(
        pl.kernel,
        mesh=mesh,
        out_type=jax.ShapeDtypeStruct((B, D), jnp.float32),
        scratch_types=[
            pltpu.VMEM((KC,), jnp.int32),
            pltpu.VMEM((_NBUF, D, _LANES), jnp.float32),
            pltpu.VMEM((KC, D), jnp.float32),
            [pltpu.SemaphoreType.DMA] * _NBUF,
        ],
        compiler_params=pltpu.CompilerParams(
            use_tc_tiling_on_sc=True, needs_layout_passes=False
        ),
    )
    def embed(idx_hbm, table_hbm, out_hbm, idx_v, blocks_v, rows_v, sems):
        wid = lax.axis_index("s") * _NC + lax.axis_index("c")
        base = wid * KC
        pltpu.sync_copy(idx_hbm.at[pl.ds(base, KC)], idx_v)
        iota16 = lax.iota(jnp.int32, 16)

        def splat(e):
            return plsc.load_gather(idx_v, [jnp.full((16,), e, jnp.int32)])

        def fetch(b, e):
            off = pl.multiple_of((splat(e) & -_LANES)[0], _LANES)
            pltpu.async_copy(
                table_hbm.at[:, pl.ds(off, _LANES)], blocks_v.at[b], sems[b]
            )

        def extract(b, e):
            lane_vec = splat(e) & (_LANES - 1)
            e_vec = jnp.full((16,), e, jnp.int32)
            for jj in range(D // 16):
                vals = plsc.load_gather(
                    blocks_v.at[b], [jj * 16 + iota16, lane_vec]
                )
                plsc.store_scatter(rows_v, [e_vec, jj * 16 + iota16], vals)

        for b in range(_NBUF):
            fetch(b, b)

        def body(g, carry):
            for b in range(_NBUF):
                e = g * _NBUF + b
                pltpu.make_async_copy(
                    table_hbm.at[:, pl.ds(0, _LANES)], blocks_v.at[b], sems[b]
                ).wait()
                extract(b, e)
                nxt = e + _NBUF

                @pl.when(nxt < KC)
                def _():
                    fetch(b, nxt)

            return carry

        lax.fori_loop(0, KC // _NBUF, body, 0)
        pltpu.sync_copy(rows_v, out_hbm.at[pl.ds(base, KC)])

    return embed


def kernel(x, weight):
    (B,) = x.shape
    V, D = weight.shape
    assert B % (_NW * _NBUF) == 0
    idx = x.astype(jnp.int32)
    out = _make_embed(B, V, D)(idx, weight.T)
    return out


# speedup vs baseline: 3.0421x; 1.2046x over previous
import functools

import jax
import jax.numpy as jnp
from jax import lax
from jax.experimental import pallas as pl
from jax.experimental.pallas import tpu as pltpu
from jax.experimental.pallas import tpu_sc as plsc

_INFO = plsc.get_sparse_core_info()
_NC = _INFO.num_cores
_NS = _INFO.num_subcores
_NW = _NC * _NS
_NBUF = 8
_LANES = 128


@functools.lru_cache(maxsize=None)
def _make_embed(B, V, D):
    KC = B // _NW
    mesh = plsc.VectorSubcoreMesh(core_axis_name="c", subcore_axis_name="s")

    @functools.partial(
        pl.kernel,
        mesh=mesh,
        out_type=jax.ShapeDtypeStruct((D, B), jnp.float32),
        scratch_types=[
            pltpu.VMEM((KC,), jnp.int32),
            pltpu.VMEM((_NBUF, D, _LANES), jnp.float32),
            pltpu.VMEM((D, KC), jnp.float32),
            [pltpu.SemaphoreType.DMA] * _NBUF,
        ],
        compiler_params=pltpu.CompilerParams(
            use_tc_tiling_on_sc=True, needs_layout_passes=False
        ),
    )
    def embed(idx_hbm, table_hbm, out_hbm, idx_v, blocks_v, cols_v, sems):
        wid = lax.axis_index("s") * _NC + lax.axis_index("c")
        base = wid * KC
        pltpu.sync_copy(idx_hbm.at[pl.ds(base, KC)], idx_v)
        iota16 = lax.iota(jnp.int32, 16)

        def splat(e):
            return plsc.load_gather(idx_v, [jnp.full((16,), e, jnp.int32)])

        def fetch(b, e):
            off = pl.multiple_of((splat(e) & -_LANES)[0], _LANES)
            pltpu.async_copy(
                table_hbm.at[:, pl.ds(off, _LANES)], blocks_v.at[b], sems[b]
            )

        def extract(b, e):
            lane_vec = splat(e) & (_LANES - 1)
            e_vec = jnp.full((16,), e, jnp.int32)
            for jj in range(D // 16):
                vals = plsc.load_gather(
                    blocks_v.at[b], [jj * 16 + iota16, lane_vec]
                )
                plsc.store_scatter(cols_v, [jj * 16 + iota16, e_vec], vals)

        for b in range(_NBUF):
            fetch(b, b)

        def body(g, carry):
            for b in range(_NBUF):
                e = g * _NBUF + b
                pltpu.make_async_copy(
                    table_hbm.at[:, pl.ds(0, _LANES)], blocks_v.at[b], sems[b]
                ).wait()
                extract(b, e)
                nxt = e + _NBUF

                @pl.when(nxt < KC)
                def _():
                    fetch(b, nxt)

            return carry

        lax.fori_loop(0, KC // _NBUF, body, 0)
        pltpu.sync_copy(cols_v, out_hbm.at[:, pl.ds(base, KC)])

    return embed


def kernel(x, weight):
    (B,) = x.shape
    V, D = weight.shape
    assert B % (_NW * _NBUF) == 0
    idx = x.astype(jnp.int32)
    out_t = _make_embed(B, V, D)(idx, weight.T)
    return out_t.T
